# baseline (speedup 1.0000x reference)
import functools
import math

import jax
import jax.numpy as jnp
from jax import lax
from jax.experimental import pallas as pl
from jax.experimental.pallas import tpu as pltpu
from jax.experimental.pallas import tpu_sc as plsc

D_MODEL = 1024
N_Q = 4096
N_K = 4096
K_TOP = 128
BM = 512
BN = 512
SEGW = 512
NSEG = (N_Q * N_K) // SEGW
NW = 32
SEG_PER_W = NSEG // NW
CAP_SEG = 128
CAP_EL = 512
L = 16


def _proj_body(x_ref, w_ref, b_ref, o_ref):
    o_ref[...] = (
        jax.lax.dot_general(
            x_ref[...], w_ref[...], (((1,), (0,)), ((), ())),
            preferred_element_type=jnp.float32,
        )
        + b_ref[...]
    )


def _project(x, W, b):
    return pl.pallas_call(
        _proj_body,
        grid=(N_Q // BM,),
        in_specs=[
            pl.BlockSpec((BM, D_MODEL), lambda i: (i, 0)),
            pl.BlockSpec((D_MODEL, D_MODEL), lambda i: (0, 0)),
            pl.BlockSpec((1, D_MODEL), lambda i: (0, 0)),
        ],
        out_specs=pl.BlockSpec((BM, D_MODEL), lambda i: (i, 0)),
        out_shape=jax.ShapeDtypeStruct((N_Q, D_MODEL), jnp.float32),
    )(x, W, b.reshape(1, D_MODEL))


def _attn_body(q_ref, k_ref, tti_ref, m_ref, ab_ref, out_ref, segmax_ref):
    j = pl.program_id(1)
    alpha = ab_ref[0, 0]
    bias = ab_ref[0, 1]
    s = jax.lax.dot_general(
        q_ref[...], k_ref[...], (((1,), (1,)), ((), ())),
        preferred_element_type=jnp.float32,
    )
    s = s * (1.0 / math.sqrt(D_MODEL))
    logits = (bias + s) - alpha * tti_ref[...]
    logits = jnp.where(m_ref[...], logits, jnp.float32(-1.0e9))
    out_ref[...] = logits
    mx = jnp.max(logits, axis=1, keepdims=True)
    col = jax.lax.broadcasted_iota(jnp.int32, (BM, N_K // BN), 1)
    prev = jnp.where(col < j, segmax_ref[...], jnp.float32(-jnp.inf))
    segmax_ref[...] = jnp.where(col == j, mx, prev)


def _attn(Q, Kp, tti, mask, alpha, bias):
    ab = jnp.stack([alpha.astype(jnp.float32), bias.astype(jnp.float32)]).reshape(1, 2)
    return pl.pallas_call(
        _attn_body,
        grid=(N_Q // BM, N_K // BN),
        in_specs=[
            pl.BlockSpec((BM, D_MODEL), lambda i, j: (i, 0)),
            pl.BlockSpec((BN, D_MODEL), lambda i, j: (j, 0)),
            pl.BlockSpec((BM, BN), lambda i, j: (i, j)),
            pl.BlockSpec((BM, BN), lambda i, j: (i, j)),
            pl.BlockSpec(memory_space=pltpu.SMEM),
        ],
        out_specs=[
            pl.BlockSpec((BM, BN), lambda i, j: (i, j)),
            pl.BlockSpec((BM, N_K // BN), lambda i, j: (i, 0)),
        ],
        out_shape=[
            jax.ShapeDtypeStruct((N_Q, N_K), jnp.float32),
            jax.ShapeDtypeStruct((N_Q, N_K // BN), jnp.float32),
        ],
    )(Q, Kp, tti, mask, ab)


def _sc_compact_body(logits_hbm, segmax_hbm, t0_hbm,
                     out_val_hbm, out_idx_hbm,
                     t0_v, segmax_v, seglist_v, data_v, oval_v, oidx_v, sem):
    wid = lax.axis_index("c") * 16 + lax.axis_index("s")
    iota = lax.iota(jnp.int32, L)
    zero = jnp.zeros((L,), jnp.int32)

    pltpu.sync_copy(t0_hbm, t0_v)
    pltpu.sync_copy(segmax_hbm.at[pl.ds(wid * SEG_PER_W, SEG_PER_W)], segmax_v)

    for c in range(CAP_SEG // L):
        seglist_v[pl.ds(c * L, L)] = zero
    for c in range(CAP_EL // L):
        oval_v[pl.ds(c * L, L)] = jnp.full((L,), -1.0e9, jnp.float32)
        oidx_v[pl.ds(c * L, L)] = zero

    t0v = t0_v[...]

    cnt = zero

    def _compact_seg(args):
        cnt, m, segid = args
        pos = cnt + plsc.cumsum(m.astype(jnp.int32)) - 1
        m2 = m & (pos < CAP_SEG)
        plsc.store_scatter(seglist_v, [pos], segid, mask=m2)
        return cnt + plsc.all_reduce_population_count(m2)

    for c in range(SEG_PER_W // L):
        v = segmax_v[pl.ds(c * L, L)]
        m = v >= t0v
        segid = wid * SEG_PER_W + c * L + iota
        cnt = lax.cond(jnp.any(m), _compact_seg, lambda a: a[0], (cnt, m, segid))

    pltpu.async_copy(logits_hbm.at[seglist_v], data_v, sem).wait()

    cnt_s = jnp.minimum(jnp.max(cnt, axis=0), CAP_SEG)

    def _compact_el(args):
        ocnt, m, vals, gidx = args
        pos = ocnt + plsc.cumsum(m.astype(jnp.int32)) - 1
        m2 = m & (pos < CAP_EL)
        plsc.store_scatter(oval_v, [pos], vals, mask=m2)
        plsc.store_scatter(oidx_v, [pos], gidx, mask=m2)
        return ocnt + plsc.all_reduce_population_count(m2)

    def seg_body(i, ocnt):
        i_b = lax.broadcast(i, (L,))
        svec = plsc.load_gather(seglist_v, [i_b])
        for c in range(SEGW // L):
            vals = plsc.load_gather(data_v, [i_b, c * L + iota])
            m = vals >= t0v
            gidx = svec * SEGW + c * L + iota
            ocnt = lax.cond(jnp.any(m), _compact_el, lambda a: a[0],
                            (ocnt, m, vals, gidx))
        return ocnt

    lax.fori_loop(0, cnt_s, seg_body, zero)

    pltpu.sync_copy(oval_v, out_val_hbm.at[wid])
    pltpu.sync_copy(oidx_v, out_idx_hbm.at[wid])


def _sc_compact(logits2d, segmax_flat, t0_vec):
    mesh = plsc.VectorSubcoreMesh(core_axis_name="c", subcore_axis_name="s")
    fn = pl.kernel(
        _sc_compact_body,
        out_type=[
            jax.ShapeDtypeStruct((NW, CAP_EL), jnp.float32),
            jax.ShapeDtypeStruct((NW, CAP_EL), jnp.int32),
        ],
        mesh=mesh,
        compiler_params=pltpu.CompilerParams(needs_layout_passes=False),
        scratch_types=[
            pltpu.VMEM((L,), jnp.float32),
            pltpu.VMEM((SEG_PER_W,), jnp.float32),
            pltpu.VMEM((CAP_SEG,), jnp.int32),
            pltpu.VMEM((CAP_SEG, SEGW), jnp.float32),
            pltpu.VMEM((CAP_EL,), jnp.float32),
            pltpu.VMEM((CAP_EL,), jnp.int32),
            pltpu.SemaphoreType.DMA,
        ],
    )
    return fn(logits2d, segmax_flat, t0_vec)


def kernel(query_set, key_set, assign_mask, pairwise_tti, Wq, bq, Wk, bk, alpha, bias, top_k):
    Q = _project(query_set, Wq, bq)
    Kp = _project(key_set, Wk, bk)
    logits, segmax = _attn(Q, Kp, pairwise_tti, assign_mask, alpha, bias)
    segmax_flat = segmax.reshape(-1)
    t0 = lax.top_k(segmax_flat, K_TOP)[0][K_TOP - 1]
    t0_vec = jnp.full((L,), t0, jnp.float32)
    logits2d = logits.reshape(NSEG, SEGW)
    cand_val, cand_idx = _sc_compact(logits2d, segmax_flat, t0_vec)
    _, bpos = lax.top_k(cand_val.reshape(-1), K_TOP)
    top_flat = cand_idx.reshape(-1)[bpos]
    i_idx = top_flat // N_K
    j_idx = top_flat % N_K
    return jnp.stack([i_idx, j_idx], axis=1)


# speedup vs baseline: 1.3929x; 1.3929x over previous
import functools
import math

import jax
import jax.numpy as jnp
from jax import lax
from jax.experimental import pallas as pl
from jax.experimental.pallas import tpu as pltpu
from jax.experimental.pallas import tpu_sc as plsc

D_MODEL = 1024
N_Q = 4096
N_K = 4096
K_TOP = 128
BM = 512
BN = 512
SEGW = 512
NSEG = (N_Q * N_K) // SEGW
NW = 32
SEG_PER_W = NSEG // NW
CAP_SEG = 128
CAP_EL = 512
L = 16


def _proj_body(x_ref, w_ref, b_ref, o_ref):
    o_ref[...] = (
        jax.lax.dot_general(
            x_ref[...], w_ref[...], (((1,), (0,)), ((), ())),
            preferred_element_type=jnp.float32,
        )
        + b_ref[...]
    )


def _project(x, W, b):
    return pl.pallas_call(
        _proj_body,
        grid=(N_Q // BM,),
        in_specs=[
            pl.BlockSpec((BM, D_MODEL), lambda i: (i, 0)),
            pl.BlockSpec((D_MODEL, D_MODEL), lambda i: (0, 0)),
            pl.BlockSpec((1, D_MODEL), lambda i: (0, 0)),
        ],
        out_specs=pl.BlockSpec((BM, D_MODEL), lambda i: (i, 0)),
        out_shape=jax.ShapeDtypeStruct((N_Q, D_MODEL), jnp.float32),
    )(x, W, b.reshape(1, D_MODEL))


def _attn_body(q_ref, k_ref, tti_ref, m_ref, ab_ref, out_ref, segmax_ref):
    j = pl.program_id(1)
    alpha = ab_ref[0, 0]
    bias = ab_ref[0, 1]
    s = jax.lax.dot_general(
        q_ref[...], k_ref[...], (((1,), (1,)), ((), ())),
        preferred_element_type=jnp.float32,
    )
    s = s * (1.0 / math.sqrt(D_MODEL))
    logits = (bias + s) - alpha * tti_ref[...]
    logits = jnp.where(m_ref[...], logits, jnp.float32(-1.0e9))
    out_ref[...] = logits
    mx = jnp.max(logits, axis=1, keepdims=True)
    col = jax.lax.broadcasted_iota(jnp.int32, (BM, N_K // BN), 1)
    prev = jnp.where(col < j, segmax_ref[...], jnp.float32(-jnp.inf))
    segmax_ref[...] = jnp.where(col == j, mx, prev)


def _attn(Q, Kp, tti, mask, alpha, bias):
    ab = jnp.stack([alpha.astype(jnp.float32), bias.astype(jnp.float32)]).reshape(1, 2)
    return pl.pallas_call(
        _attn_body,
        grid=(N_Q // BM, N_K // BN),
        in_specs=[
            pl.BlockSpec((BM, D_MODEL), lambda i, j: (i, 0)),
            pl.BlockSpec((BN, D_MODEL), lambda i, j: (j, 0)),
            pl.BlockSpec((BM, BN), lambda i, j: (i, j)),
            pl.BlockSpec((BM, BN), lambda i, j: (i, j)),
            pl.BlockSpec(memory_space=pltpu.SMEM),
        ],
        out_specs=[
            pl.BlockSpec((BM, BN), lambda i, j: (i, j)),
            pl.BlockSpec((BM, N_K // BN), lambda i, j: (i, 0)),
        ],
        out_shape=[
            jax.ShapeDtypeStruct((N_Q, N_K), jnp.float32),
            jax.ShapeDtypeStruct((N_Q, N_K // BN), jnp.float32),
        ],
    )(Q, Kp, tti, mask, ab)


def _sc_compact_body(logits_hbm, segmax_hbm, t0_hbm,
                     out_val_hbm, out_idx_hbm,
                     t0_v, segmax_v, seglist_v, data_v, oval_v, oidx_v, sem):
    wid = lax.axis_index("c") * 16 + lax.axis_index("s")
    iota = lax.iota(jnp.int32, L)
    zero = jnp.zeros((L,), jnp.int32)

    pltpu.sync_copy(t0_hbm, t0_v)
    pltpu.sync_copy(segmax_hbm.at[pl.ds(wid * SEG_PER_W, SEG_PER_W)], segmax_v)

    for c in range(CAP_SEG // L):
        seglist_v[pl.ds(c * L, L)] = zero
    for c in range(CAP_EL // L):
        oval_v[pl.ds(c * L, L)] = jnp.full((L,), -1.0e9, jnp.float32)
        oidx_v[pl.ds(c * L, L)] = zero

    t0v = t0_v[...]

    cnt = zero

    def _compact_seg(args):
        cnt, m, segid = args
        pos = cnt + plsc.cumsum(m.astype(jnp.int32)) - 1
        m2 = m & (pos < CAP_SEG)
        plsc.store_scatter(seglist_v, [pos], segid, mask=m2)
        return cnt + plsc.all_reduce_population_count(m2)

    for c in range(SEG_PER_W // L):
        v = segmax_v[pl.ds(c * L, L)]
        m = v >= t0v
        segid = wid * SEG_PER_W + c * L + iota
        cnt = lax.cond(jnp.any(m), _compact_seg, lambda a: a[0], (cnt, m, segid))

    cnt_s = jnp.minimum(jnp.max(cnt, axis=0), CAP_SEG)
    GCH = CAP_SEG // L
    for g in range(GCH):
        @pl.when(g * L < cnt_s)
        def _():
            pltpu.async_copy(
                logits_hbm.at[seglist_v.at[pl.ds(g * L, L)]],
                data_v.at[pl.ds(g * L, L)], sem)
    for g in range(GCH):
        @pl.when(g * L < cnt_s)
        def _():
            pltpu.make_async_copy(
                logits_hbm.at[seglist_v.at[pl.ds(g * L, L)]],
                data_v.at[pl.ds(g * L, L)], sem).wait()

    def _compact_el(args):
        ocnt, m, vals, gidx = args
        pos = ocnt + plsc.cumsum(m.astype(jnp.int32)) - 1
        m2 = m & (pos < CAP_EL)
        plsc.store_scatter(oval_v, [pos], vals, mask=m2)
        plsc.store_scatter(oidx_v, [pos], gidx, mask=m2)
        return ocnt + plsc.all_reduce_population_count(m2)

    def seg_body(i, ocnt):
        i_b = lax.broadcast(i, (L,))
        svec = plsc.load_gather(seglist_v, [i_b])
        for c in range(SEGW // L):
            vals = plsc.load_gather(data_v, [i_b, c * L + iota])
            m = vals >= t0v
            gidx = svec * SEGW + c * L + iota
            ocnt = lax.cond(jnp.any(m), _compact_el, lambda a: a[0],
                            (ocnt, m, vals, gidx))
        return ocnt

    lax.fori_loop(0, cnt_s, seg_body, zero)

    pltpu.sync_copy(oval_v, out_val_hbm.at[wid])
    pltpu.sync_copy(oidx_v, out_idx_hbm.at[wid])


def _sc_compact(logits2d, segmax_flat, t0_vec):
    mesh = plsc.VectorSubcoreMesh(core_axis_name="c", subcore_axis_name="s")
    fn = pl.kernel(
        _sc_compact_body,
        out_type=[
            jax.ShapeDtypeStruct((NW, CAP_EL), jnp.float32),
            jax.ShapeDtypeStruct((NW, CAP_EL), jnp.int32),
        ],
        mesh=mesh,
        compiler_params=pltpu.CompilerParams(needs_layout_passes=False),
        scratch_types=[
            pltpu.VMEM((L,), jnp.float32),
            pltpu.VMEM((SEG_PER_W,), jnp.float32),
            pltpu.VMEM((CAP_SEG,), jnp.int32),
            pltpu.VMEM((CAP_SEG, SEGW), jnp.float32),
            pltpu.VMEM((CAP_EL,), jnp.float32),
            pltpu.VMEM((CAP_EL,), jnp.int32),
            pltpu.SemaphoreType.DMA,
        ],
    )
    return fn(logits2d, segmax_flat, t0_vec)


def kernel(query_set, key_set, assign_mask, pairwise_tti, Wq, bq, Wk, bk, alpha, bias, top_k):
    Q = _project(query_set, Wq, bq)
    Kp = _project(key_set, Wk, bk)
    logits, segmax = _attn(Q, Kp, pairwise_tti, assign_mask, alpha, bias)
    segmax_flat = segmax.reshape(-1)
    t0 = lax.top_k(segmax_flat, K_TOP)[0][K_TOP - 1]
    t0_vec = jnp.full((L,), t0, jnp.float32)
    logits2d = logits.reshape(NSEG, SEGW)
    cand_val, cand_idx = _sc_compact(logits2d, segmax_flat, t0_vec)
    _, bpos = lax.top_k(cand_val.reshape(-1), K_TOP)
    top_flat = cand_idx.reshape(-1)[bpos]
    i_idx = top_flat // N_K
    j_idx = top_flat % N_K
    return jnp.stack([i_idx, j_idx], axis=1)


# speedup vs baseline: 1.3949x; 1.0014x over previous
import math

import jax
import jax.numpy as jnp
from jax import lax
from jax.experimental import pallas as pl
from jax.experimental.pallas import tpu as pltpu
from jax.experimental.pallas import tpu_sc as plsc

D_MODEL = 1024
N_Q = 4096
N_K = 4096
K_TOP = 128
BM = 512
BN = 512
SEGW = 512
NSEG = (N_Q * N_K) // SEGW
NW = 32
SEG_PER_W = NSEG // NW
CAP_SEG = 128
CAP_EL = 512
L = 16


def _proj_body(x_ref, w_ref, b_ref, o_ref):
    o_ref[...] = (
        jax.lax.dot_general(
            x_ref[...], w_ref[...], (((1,), (0,)), ((), ())),
            preferred_element_type=jnp.float32,
        )
        + b_ref[...]
    )


def _project(x, W, b):
    return pl.pallas_call(
        _proj_body,
        grid=(N_Q // BM,),
        in_specs=[
            pl.BlockSpec((BM, D_MODEL), lambda i: (i, 0)),
            pl.BlockSpec((D_MODEL, D_MODEL), lambda i: (0, 0)),
            pl.BlockSpec((1, D_MODEL), lambda i: (0, 0)),
        ],
        out_specs=pl.BlockSpec((BM, D_MODEL), lambda i: (i, 0)),
        out_shape=jax.ShapeDtypeStruct((N_Q, D_MODEL), jnp.float32),
    )(x, W, b.reshape(1, D_MODEL))


def _attn_body(q_ref, k_ref, tti_ref, m_ref, ab_ref, out_ref, segmax_ref):
    j = pl.program_id(1)
    alpha = ab_ref[0, 0]
    bias = ab_ref[0, 1]
    s = jax.lax.dot_general(
        q_ref[...], k_ref[...], (((1,), (1,)), ((), ())),
        preferred_element_type=jnp.float32,
    )
    s = s * (1.0 / math.sqrt(D_MODEL))
    logits = (bias + s) - alpha * tti_ref[...]
    logits = jnp.where(m_ref[...], logits, jnp.float32(-1.0e9))
    out_ref[...] = logits
    mx = jnp.max(logits, axis=1, keepdims=True)
    col = jax.lax.broadcasted_iota(jnp.int32, (BM, N_K // BN), 1)
    prev = jnp.where(col < j, segmax_ref[...], jnp.float32(-jnp.inf))
    segmax_ref[...] = jnp.where(col == j, mx, prev)


def _attn(Q, Kp, tti, mask, alpha, bias):
    ab = jnp.stack([alpha.astype(jnp.float32), bias.astype(jnp.float32)]).reshape(1, 2)
    return pl.pallas_call(
        _attn_body,
        grid=(N_Q // BM, N_K // BN),
        in_specs=[
            pl.BlockSpec((BM, D_MODEL), lambda i, j: (i, 0)),
            pl.BlockSpec((BN, D_MODEL), lambda i, j: (j, 0)),
            pl.BlockSpec((BM, BN), lambda i, j: (i, j)),
            pl.BlockSpec((BM, BN), lambda i, j: (i, j)),
            pl.BlockSpec(memory_space=pltpu.SMEM),
        ],
        out_specs=[
            pl.BlockSpec((BM, BN), lambda i, j: (i, j)),
            pl.BlockSpec((BM, N_K // BN), lambda i, j: (i, 0)),
        ],
        out_shape=[
            jax.ShapeDtypeStruct((N_Q, N_K), jnp.float32),
            jax.ShapeDtypeStruct((N_Q, N_K // BN), jnp.float32),
        ],
    )(Q, Kp, tti, mask, ab)


def _sc_compact_body(logits_hbm, segmax_hbm, t0_hbm,
                     out_val_hbm, out_idx_hbm,
                     t0_v, segmax_v, seglist_v, data_v, oval_v, oidx_v, sem):
    wid = lax.axis_index("c") * 16 + lax.axis_index("s")
    iota = lax.iota(jnp.int32, L)
    zero = jnp.zeros((L,), jnp.int32)

    pltpu.sync_copy(t0_hbm, t0_v)
    pltpu.sync_copy(segmax_hbm.at[pl.ds(wid * SEG_PER_W, SEG_PER_W)], segmax_v)

    for c in range(CAP_SEG // L):
        seglist_v[pl.ds(c * L, L)] = zero
    for c in range(CAP_EL // L):
        oval_v[pl.ds(c * L, L)] = jnp.full((L,), -1.0e9, jnp.float32)
        oidx_v[pl.ds(c * L, L)] = zero

    t0v = t0_v[...]

    cnt = zero

    def _compact_seg(args):
        cnt, m, segid = args
        pos = cnt + plsc.cumsum(m.astype(jnp.int32)) - 1
        m2 = m & (pos < CAP_SEG)
        plsc.store_scatter(seglist_v, [pos], segid, mask=m2)
        return cnt + plsc.all_reduce_population_count(m2)

    for c in range(SEG_PER_W // L):
        v = segmax_v[pl.ds(c * L, L)]
        m = v >= t0v
        segid = wid * SEG_PER_W + c * L + iota
        cnt = lax.cond(jnp.any(m), _compact_seg, lambda a: a[0], (cnt, m, segid))

    cnt_s = jnp.minimum(jnp.max(cnt, axis=0), CAP_SEG)
    GCH = CAP_SEG // L
    for g in range(GCH):
        @pl.when(g * L < cnt_s)
        def _():
            pltpu.async_copy(
                logits_hbm.at[seglist_v.at[pl.ds(g * L, L)]],
                data_v.at[pl.ds(g * L, L)], sem)
    for g in range(GCH):
        @pl.when(g * L < cnt_s)
        def _():
            pltpu.make_async_copy(
                logits_hbm.at[seglist_v.at[pl.ds(g * L, L)]],
                data_v.at[pl.ds(g * L, L)], sem).wait()

    def _compact_el(args):
        ocnt, m, vals, gidx = args
        pos = ocnt + plsc.cumsum(m.astype(jnp.int32)) - 1
        m2 = m & (pos < CAP_EL)
        plsc.store_scatter(oval_v, [pos], vals, mask=m2)
        plsc.store_scatter(oidx_v, [pos], gidx, mask=m2)
        return ocnt + plsc.all_reduce_population_count(m2)

    def seg_body(i, ocnt):
        i_b = lax.broadcast(i, (L,))
        svec = plsc.load_gather(seglist_v, [i_b])
        for c in range(SEGW // L):
            vals = plsc.load_gather(data_v, [i_b, c * L + iota])
            m = vals >= t0v
            gidx = svec * SEGW + c * L + iota
            ocnt = lax.cond(jnp.any(m), _compact_el, lambda a: a[0],
                            (ocnt, m, vals, gidx))
        return ocnt

    lax.fori_loop(0, cnt_s, seg_body, zero)

    pltpu.sync_copy(oval_v, out_val_hbm.at[wid])
    pltpu.sync_copy(oidx_v, out_idx_hbm.at[wid])


def _sc_compact(logits2d, segmax_flat, t0_vec):
    mesh = plsc.VectorSubcoreMesh(core_axis_name="c", subcore_axis_name="s")
    fn = pl.kernel(
        _sc_compact_body,
        out_type=[
            jax.ShapeDtypeStruct((NW, CAP_EL), jnp.float32),
            jax.ShapeDtypeStruct((NW, CAP_EL), jnp.int32),
        ],
        mesh=mesh,
        compiler_params=pltpu.CompilerParams(needs_layout_passes=False),
        scratch_types=[
            pltpu.VMEM((L,), jnp.float32),
            pltpu.VMEM((SEG_PER_W,), jnp.float32),
            pltpu.VMEM((CAP_SEG,), jnp.int32),
            pltpu.VMEM((CAP_SEG, SEGW), jnp.float32),
            pltpu.VMEM((CAP_EL,), jnp.float32),
            pltpu.VMEM((CAP_EL,), jnp.int32),
            pltpu.SemaphoreType.DMA,
        ],
    )
    return fn(logits2d, segmax_flat, t0_vec)


def kernel(query_set, key_set, assign_mask, pairwise_tti, Wq, bq, Wk, bk, alpha, bias, top_k):
    Q = _project(query_set, Wq, bq)
    Kp = _project(key_set, Wk, bk)
    logits, segmax = _attn(Q, Kp, pairwise_tti, assign_mask, alpha, bias)
    segmax_flat = segmax.reshape(-1)
    t0 = lax.top_k(segmax_flat, K_TOP)[0][K_TOP - 1]
    t0_vec = jnp.full((L,), t0, jnp.float32)
    logits2d = logits.reshape(NSEG, SEGW)
    cand_val, cand_idx = _sc_compact(logits2d, segmax_flat, t0_vec)
    _, bpos = lax.top_k(cand_val.reshape(-1), K_TOP)
    top_flat = cand_idx.reshape(-1)[bpos]
    i_idx = top_flat // N_K
    j_idx = top_flat % N_K
    return jnp.stack([i_idx, j_idx], axis=1)


# speedup vs baseline: 1.5341x; 1.0998x over previous
import math

import jax
import jax.numpy as jnp
from jax import lax
from jax.experimental import pallas as pl
from jax.experimental.pallas import tpu as pltpu
from jax.experimental.pallas import tpu_sc as plsc

D_MODEL = 1024
N_Q = 4096
N_K = 4096
K_TOP = 128
BM = 512
BN = 512
SEGW = 512
NSEG = (N_Q * N_K) // SEGW
NW = 32
SEG_PER_W = NSEG // NW
CAP_SEG = 128
CAP_EL = 512
L = 16


def _proj_body(x_ref, w_ref, b_ref, o_ref):
    o_ref[...] = (
        jax.lax.dot_general(
            x_ref[...], w_ref[...], (((1,), (0,)), ((), ())),
            preferred_element_type=jnp.float32,
        )
        + b_ref[...]
    )


def _project(x, W, b):
    return pl.pallas_call(
        _proj_body,
        grid=(N_Q // BM,),
        in_specs=[
            pl.BlockSpec((BM, D_MODEL), lambda i: (i, 0)),
            pl.BlockSpec((D_MODEL, D_MODEL), lambda i: (0, 0)),
            pl.BlockSpec((1, D_MODEL), lambda i: (0, 0)),
        ],
        out_specs=pl.BlockSpec((BM, D_MODEL), lambda i: (i, 0)),
        out_shape=jax.ShapeDtypeStruct((N_Q, D_MODEL), jnp.float32),
    )(x, W, b.reshape(1, D_MODEL))


def _attn_body(q_ref, k_ref, tti_ref, m_ref, ab_ref, out_ref, segmax_ref):
    j = pl.program_id(1)
    alpha = ab_ref[0, 0]
    bias = ab_ref[0, 1]
    s = jax.lax.dot_general(
        q_ref[...], k_ref[...], (((1,), (1,)), ((), ())),
        preferred_element_type=jnp.float32,
    )
    s = s * (1.0 / math.sqrt(D_MODEL))
    logits = (bias + s) - alpha * tti_ref[...]
    logits = jnp.where(m_ref[...], logits, jnp.float32(-1.0e9))
    out_ref[...] = logits
    mx = jnp.max(logits, axis=1, keepdims=True)
    col = jax.lax.broadcasted_iota(jnp.int32, (logits.shape[0], N_K // BN), 1)
    prev = jnp.where(col < j, segmax_ref[...], jnp.float32(-jnp.inf))
    segmax_ref[...] = jnp.where(col == j, mx, prev)


def _attn(Q, Kp, tti, mask, alpha, bias):
    ab = jnp.stack([alpha.astype(jnp.float32), bias.astype(jnp.float32)]).reshape(1, 2)
    bm = 1024
    return pl.pallas_call(
        _attn_body,
        grid=(N_Q // bm, N_K // BN),
        in_specs=[
            pl.BlockSpec((bm, D_MODEL), lambda i, j: (i, 0)),
            pl.BlockSpec((BN, D_MODEL), lambda i, j: (j, 0)),
            pl.BlockSpec((bm, BN), lambda i, j: (i, j)),
            pl.BlockSpec((bm, BN), lambda i, j: (i, j)),
            pl.BlockSpec(memory_space=pltpu.SMEM),
        ],
        out_specs=[
            pl.BlockSpec((bm, BN), lambda i, j: (i, j)),
            pl.BlockSpec((bm, N_K // BN), lambda i, j: (i, 0)),
        ],
        out_shape=[
            jax.ShapeDtypeStruct((N_Q, N_K), jnp.float32),
            jax.ShapeDtypeStruct((N_Q, N_K // BN), jnp.float32),
        ],
    )(Q, Kp, tti, mask, ab)


def _sc_compact_body(logits_hbm, segmax_hbm, t0_hbm,
                     out_val_hbm, out_idx_hbm,
                     t0_v, segmax_v, seglist_v, data_v, oval_v, oidx_v, sem):
    wid = lax.axis_index("c") * 16 + lax.axis_index("s")
    iota = lax.iota(jnp.int32, L)
    zero = jnp.zeros((L,), jnp.int32)

    pltpu.sync_copy(t0_hbm, t0_v)
    pltpu.sync_copy(segmax_hbm.at[pl.ds(wid * SEG_PER_W, SEG_PER_W)], segmax_v)

    for c in range(CAP_SEG // L):
        seglist_v[pl.ds(c * L, L)] = zero
    for c in range(CAP_EL // L):
        oval_v[pl.ds(c * L, L)] = jnp.full((L,), -1.0e9, jnp.float32)
        oidx_v[pl.ds(c * L, L)] = zero

    t0v = t0_v[...]

    cnt = zero

    def _compact_seg(args):
        cnt, m, segid = args
        pos = cnt + plsc.cumsum(m.astype(jnp.int32)) - 1
        m2 = m & (pos < CAP_SEG)
        plsc.store_scatter(seglist_v, [pos], segid, mask=m2)
        return cnt + plsc.all_reduce_population_count(m2)

    for c in range(SEG_PER_W // L):
        v = segmax_v[pl.ds(c * L, L)]
        m = v >= t0v
        segid = wid * SEG_PER_W + c * L + iota
        cnt = lax.cond(jnp.any(m), _compact_seg, lambda a: a[0], (cnt, m, segid))

    cnt_s = jnp.minimum(jnp.max(cnt, axis=0), CAP_SEG)
    GCH = CAP_SEG // L
    for g in range(GCH):
        @pl.when(g * L < cnt_s)
        def _():
            pltpu.async_copy(
                logits_hbm.at[seglist_v.at[pl.ds(g * L, L)]],
                data_v.at[pl.ds(g * L, L)], sem)
    for g in range(GCH):
        @pl.when(g * L < cnt_s)
        def _():
            pltpu.make_async_copy(
                logits_hbm.at[seglist_v.at[pl.ds(g * L, L)]],
                data_v.at[pl.ds(g * L, L)], sem).wait()

    def _compact_el(args):
        ocnt, m, vals, gidx = args
        pos = ocnt + plsc.cumsum(m.astype(jnp.int32)) - 1
        m2 = m & (pos < CAP_EL)
        plsc.store_scatter(oval_v, [pos], vals, mask=m2)
        plsc.store_scatter(oidx_v, [pos], gidx, mask=m2)
        return ocnt + plsc.all_reduce_population_count(m2)

    def seg_body(i, ocnt):
        i_b = lax.broadcast(i, (L,))
        svec = plsc.load_gather(seglist_v, [i_b])
        for c in range(SEGW // L):
            vals = plsc.load_gather(data_v, [i_b, c * L + iota])
            m = vals >= t0v
            gidx = svec * SEGW + c * L + iota
            ocnt = lax.cond(jnp.any(m), _compact_el, lambda a: a[0],
                            (ocnt, m, vals, gidx))
        return ocnt

    lax.fori_loop(0, cnt_s, seg_body, zero)

    pltpu.sync_copy(oval_v, out_val_hbm.at[wid])
    pltpu.sync_copy(oidx_v, out_idx_hbm.at[wid])


def _sc_compact(logits2d, segmax_flat, t0_vec):
    mesh = plsc.VectorSubcoreMesh(core_axis_name="c", subcore_axis_name="s")
    fn = pl.kernel(
        _sc_compact_body,
        out_type=[
            jax.ShapeDtypeStruct((NW, CAP_EL), jnp.float32),
            jax.ShapeDtypeStruct((NW, CAP_EL), jnp.int32),
        ],
        mesh=mesh,
        compiler_params=pltpu.CompilerParams(needs_layout_passes=False),
        scratch_types=[
            pltpu.VMEM((L,), jnp.float32),
            pltpu.VMEM((SEG_PER_W,), jnp.float32),
            pltpu.VMEM((CAP_SEG,), jnp.int32),
            pltpu.VMEM((CAP_SEG, SEGW), jnp.float32),
            pltpu.VMEM((CAP_EL,), jnp.float32),
            pltpu.VMEM((CAP_EL,), jnp.int32),
            pltpu.SemaphoreType.DMA,
        ],
    )
    return fn(logits2d, segmax_flat, t0_vec)


def kernel(query_set, key_set, assign_mask, pairwise_tti, Wq, bq, Wk, bk, alpha, bias, top_k):
    Q = _project(query_set, Wq, bq)
    Kp = _project(key_set, Wk, bk)
    logits, segmax = _attn(Q, Kp, pairwise_tti, assign_mask, alpha, bias)
    segmax_flat = segmax.reshape(-1)
    t0 = lax.top_k(segmax_flat, K_TOP)[0][K_TOP - 1]
    t0_vec = jnp.full((L,), t0, jnp.float32)
    logits2d = logits.reshape(NSEG, SEGW)
    cand_val, cand_idx = _sc_compact(logits2d, segmax_flat, t0_vec)
    _, bpos = lax.top_k(cand_val.reshape(-1), K_TOP)
    top_flat = cand_idx.reshape(-1)[bpos]
    i_idx = top_flat // N_K
    j_idx = top_flat % N_K
    return jnp.stack([i_idx, j_idx], axis=1)


# speedup vs baseline: 1.5913x; 1.0373x over previous
import math

import jax
import jax.numpy as jnp
from jax import lax
from jax.experimental import pallas as pl
from jax.experimental.pallas import tpu as pltpu
from jax.experimental.pallas import tpu_sc as plsc

D_MODEL = 1024
N_Q = 4096
N_K = 4096
K_TOP = 128
BM = 512
BN = 512
SEGW = 512
NSEG = (N_Q * N_K) // SEGW
NW = 32
SEG_PER_W = NSEG // NW
CAP_SEG = 128
CAP_EL = 512
L = 16


def _proj_body(x_ref, w_ref, b_ref, o_ref):
    o_ref[...] = (
        jax.lax.dot_general(
            x_ref[...], w_ref[...], (((1,), (0,)), ((), ())),
            preferred_element_type=jnp.float32,
        )
        + b_ref[...]
    )


def _project(x, W, b):
    return pl.pallas_call(
        _proj_body,
        grid=(N_Q // BM,),
        in_specs=[
            pl.BlockSpec((BM, D_MODEL), lambda i: (i, 0)),
            pl.BlockSpec((D_MODEL, D_MODEL), lambda i: (0, 0)),
            pl.BlockSpec((1, D_MODEL), lambda i: (0, 0)),
        ],
        out_specs=pl.BlockSpec((BM, D_MODEL), lambda i: (i, 0)),
        out_shape=jax.ShapeDtypeStruct((N_Q, D_MODEL), jnp.float32),
    )(x, W, b.reshape(1, D_MODEL))


def _attn_body(q_ref, k_ref, tti_ref, m_ref, ab_ref, out_ref, segmax_ref):
    j = pl.program_id(1)
    alpha = ab_ref[0, 0]
    bias = ab_ref[0, 1]
    s = jax.lax.dot_general(
        q_ref[...], k_ref[...], (((1,), (1,)), ((), ())),
        preferred_element_type=jnp.float32,
    )
    s = s * (1.0 / math.sqrt(D_MODEL))
    logits = (bias + s) - alpha * tti_ref[...]
    logits = jnp.where(m_ref[...], logits, jnp.float32(-1.0e9))
    out_ref[...] = logits
    mx = jnp.max(logits, axis=1, keepdims=True)
    col = jax.lax.broadcasted_iota(jnp.int32, (logits.shape[0], N_K // BN), 1)
    prev = jnp.where(col < j, segmax_ref[...], jnp.float32(-jnp.inf))
    segmax_ref[...] = jnp.where(col == j, mx, prev)


def _attn(Q, Kp, tti, mask, alpha, bias):
    ab = jnp.stack([alpha.astype(jnp.float32), bias.astype(jnp.float32)]).reshape(1, 2)
    bm = 2048
    return pl.pallas_call(
        _attn_body,
        grid=(N_Q // bm, N_K // BN),
        in_specs=[
            pl.BlockSpec((bm, D_MODEL), lambda i, j: (i, 0)),
            pl.BlockSpec((BN, D_MODEL), lambda i, j: (j, 0)),
            pl.BlockSpec((bm, BN), lambda i, j: (i, j)),
            pl.BlockSpec((bm, BN), lambda i, j: (i, j)),
            pl.BlockSpec(memory_space=pltpu.SMEM),
        ],
        out_specs=[
            pl.BlockSpec((bm, BN), lambda i, j: (i, j)),
            pl.BlockSpec((bm, N_K // BN), lambda i, j: (i, 0)),
        ],
        out_shape=[
            jax.ShapeDtypeStruct((N_Q, N_K), jnp.float32),
            jax.ShapeDtypeStruct((N_Q, N_K // BN), jnp.float32),
        ],
    )(Q, Kp, tti, mask, ab)


def _sc_compact_body(logits_hbm, segmax_hbm, t0_hbm,
                     out_val_hbm, out_idx_hbm,
                     t0_v, segmax_v, seglist_v, data_v, oval_v, oidx_v, sem):
    wid = lax.axis_index("c") * 16 + lax.axis_index("s")
    iota = lax.iota(jnp.int32, L)
    zero = jnp.zeros((L,), jnp.int32)

    pltpu.sync_copy(t0_hbm, t0_v)
    pltpu.sync_copy(segmax_hbm.at[pl.ds(wid * SEG_PER_W, SEG_PER_W)], segmax_v)

    for c in range(CAP_SEG // L):
        seglist_v[pl.ds(c * L, L)] = zero
    for c in range(CAP_EL // L):
        oval_v[pl.ds(c * L, L)] = jnp.full((L,), -1.0e9, jnp.float32)
        oidx_v[pl.ds(c * L, L)] = zero

    t0v = t0_v[...]

    cnt = zero

    def _compact_seg(args):
        cnt, m, segid = args
        pos = cnt + plsc.cumsum(m.astype(jnp.int32)) - 1
        m2 = m & (pos < CAP_SEG)
        plsc.store_scatter(seglist_v, [pos], segid, mask=m2)
        return cnt + plsc.all_reduce_population_count(m2)

    for c in range(SEG_PER_W // L):
        v = segmax_v[pl.ds(c * L, L)]
        m = v >= t0v
        segid = wid * SEG_PER_W + c * L + iota
        cnt = lax.cond(jnp.any(m), _compact_seg, lambda a: a[0], (cnt, m, segid))

    cnt_s = jnp.minimum(jnp.max(cnt, axis=0), CAP_SEG)
    GCH = CAP_SEG // L
    for g in range(GCH):
        @pl.when(g * L < cnt_s)
        def _():
            pltpu.async_copy(
                logits_hbm.at[seglist_v.at[pl.ds(g * L, L)]],
                data_v.at[pl.ds(g * L, L)], sem)
    for g in range(GCH):
        @pl.when(g * L < cnt_s)
        def _():
            pltpu.make_async_copy(
                logits_hbm.at[seglist_v.at[pl.ds(g * L, L)]],
                data_v.at[pl.ds(g * L, L)], sem).wait()

    def _compact_el(args):
        ocnt, m, vals, gidx = args
        pos = ocnt + plsc.cumsum(m.astype(jnp.int32)) - 1
        m2 = m & (pos < CAP_EL)
        plsc.store_scatter(oval_v, [pos], vals, mask=m2)
        plsc.store_scatter(oidx_v, [pos], gidx, mask=m2)
        return ocnt + plsc.all_reduce_population_count(m2)

    def seg_body(i, ocnt):
        i_b = lax.broadcast(i, (L,))
        svec = plsc.load_gather(seglist_v, [i_b])
        for c in range(SEGW // L):
            vals = plsc.load_gather(data_v, [i_b, c * L + iota])
            m = vals >= t0v
            gidx = svec * SEGW + c * L + iota
            ocnt = lax.cond(jnp.any(m), _compact_el, lambda a: a[0],
                            (ocnt, m, vals, gidx))
        return ocnt

    lax.fori_loop(0, cnt_s, seg_body, zero)

    pltpu.sync_copy(oval_v, out_val_hbm.at[wid])
    pltpu.sync_copy(oidx_v, out_idx_hbm.at[wid])


def _sc_compact(logits2d, segmax_flat, t0_vec):
    mesh = plsc.VectorSubcoreMesh(core_axis_name="c", subcore_axis_name="s")
    fn = pl.kernel(
        _sc_compact_body,
        out_type=[
            jax.ShapeDtypeStruct((NW, CAP_EL), jnp.float32),
            jax.ShapeDtypeStruct((NW, CAP_EL), jnp.int32),
        ],
        mesh=mesh,
        compiler_params=pltpu.CompilerParams(needs_layout_passes=False),
        scratch_types=[
            pltpu.VMEM((L,), jnp.float32),
            pltpu.VMEM((SEG_PER_W,), jnp.float32),
            pltpu.VMEM((CAP_SEG,), jnp.int32),
            pltpu.VMEM((CAP_SEG, SEGW), jnp.float32),
            pltpu.VMEM((CAP_EL,), jnp.float32),
            pltpu.VMEM((CAP_EL,), jnp.int32),
            pltpu.SemaphoreType.DMA,
        ],
    )
    return fn(logits2d, segmax_flat, t0_vec)


def kernel(query_set, key_set, assign_mask, pairwise_tti, Wq, bq, Wk, bk, alpha, bias, top_k):
    Q = _project(query_set, Wq, bq)
    Kp = _project(key_set, Wk, bk)
    logits, segmax = _attn(Q, Kp, pairwise_tti, assign_mask, alpha, bias)
    segmax_flat = segmax.reshape(-1)
    t0 = lax.top_k(segmax_flat, K_TOP)[0][K_TOP - 1]
    t0_vec = jnp.full((L,), t0, jnp.float32)
    logits2d = logits.reshape(NSEG, SEGW)
    cand_val, cand_idx = _sc_compact(logits2d, segmax_flat, t0_vec)
    _, bpos = lax.top_k(cand_val.reshape(-1), K_TOP)
    top_flat = cand_idx.reshape(-1)[bpos]
    i_idx = top_flat // N_K
    j_idx = top_flat % N_K
    return jnp.stack([i_idx, j_idx], axis=1)


# speedup vs baseline: 1.5932x; 1.0012x over previous
import math

import jax
import jax.numpy as jnp
from jax import lax
from jax.experimental import pallas as pl
from jax.experimental.pallas import tpu as pltpu
from jax.experimental.pallas import tpu_sc as plsc

D_MODEL = 1024
N_Q = 4096
N_K = 4096
K_TOP = 128
BM = 512
BN = 512
SEGW = 512
NSEG = (N_Q * N_K) // SEGW
NW = 32
SEG_PER_W = NSEG // NW
CAP_SEG = 128
CAP_EL = 512
L = 16


def _proj_body(x_ref, w_ref, b_ref, o_ref):
    o_ref[...] = (
        jax.lax.dot_general(
            x_ref[...], w_ref[...], (((1,), (0,)), ((), ())),
            preferred_element_type=jnp.float32,
        )
        + b_ref[...]
    )


def _project(x, W, b):
    BM_P = 2048
    return pl.pallas_call(
        _proj_body,
        grid=(N_Q // BM_P,),
        in_specs=[
            pl.BlockSpec((BM_P, D_MODEL), lambda i: (i, 0)),
            pl.BlockSpec((D_MODEL, D_MODEL), lambda i: (0, 0)),
            pl.BlockSpec((1, D_MODEL), lambda i: (0, 0)),
        ],
        out_specs=pl.BlockSpec((BM_P, D_MODEL), lambda i: (i, 0)),
        out_shape=jax.ShapeDtypeStruct((N_Q, D_MODEL), jnp.float32),
    )(x, W, b.reshape(1, D_MODEL))


def _attn_body(q_ref, k_ref, tti_ref, m_ref, ab_ref, out_ref, segmax_ref):
    j = pl.program_id(1)
    alpha = ab_ref[0, 0]
    bias = ab_ref[0, 1]
    s = jax.lax.dot_general(
        q_ref[...], k_ref[...], (((1,), (1,)), ((), ())),
        preferred_element_type=jnp.float32,
    )
    s = s * (1.0 / math.sqrt(D_MODEL))
    logits = (bias + s) - alpha * tti_ref[...]
    logits = jnp.where(m_ref[...], logits, jnp.float32(-1.0e9))
    out_ref[...] = logits
    mx = jnp.max(logits, axis=1, keepdims=True)
    col = jax.lax.broadcasted_iota(jnp.int32, (logits.shape[0], N_K // BN), 1)
    prev = jnp.where(col < j, segmax_ref[...], jnp.float32(-jnp.inf))
    segmax_ref[...] = jnp.where(col == j, mx, prev)


def _attn(Q, Kp, tti, mask, alpha, bias):
    ab = jnp.stack([alpha.astype(jnp.float32), bias.astype(jnp.float32)]).reshape(1, 2)
    bm = 2048
    return pl.pallas_call(
        _attn_body,
        grid=(N_Q // bm, N_K // BN),
        in_specs=[
            pl.BlockSpec((bm, D_MODEL), lambda i, j: (i, 0)),
            pl.BlockSpec((BN, D_MODEL), lambda i, j: (j, 0)),
            pl.BlockSpec((bm, BN), lambda i, j: (i, j)),
            pl.BlockSpec((bm, BN), lambda i, j: (i, j)),
            pl.BlockSpec(memory_space=pltpu.SMEM),
        ],
        out_specs=[
            pl.BlockSpec((bm, BN), lambda i, j: (i, j)),
            pl.BlockSpec((bm, N_K // BN), lambda i, j: (i, 0)),
        ],
        out_shape=[
            jax.ShapeDtypeStruct((N_Q, N_K), jnp.float32),
            jax.ShapeDtypeStruct((N_Q, N_K // BN), jnp.float32),
        ],
    )(Q, Kp, tti, mask, ab)


def _sc_compact_body(logits_hbm, segmax_hbm, t0_hbm,
                     out_val_hbm, out_idx_hbm,
                     t0_v, segmax_v, seglist_v, data_v, oval_v, oidx_v, sem):
    wid = lax.axis_index("c") * 16 + lax.axis_index("s")
    iota = lax.iota(jnp.int32, L)
    zero = jnp.zeros((L,), jnp.int32)

    pltpu.sync_copy(t0_hbm, t0_v)
    pltpu.sync_copy(segmax_hbm.at[pl.ds(wid * SEG_PER_W, SEG_PER_W)], segmax_v)

    for c in range(CAP_SEG // L):
        seglist_v[pl.ds(c * L, L)] = zero
    for c in range(CAP_EL // L):
        oval_v[pl.ds(c * L, L)] = jnp.full((L,), -1.0e9, jnp.float32)
        oidx_v[pl.ds(c * L, L)] = zero

    t0v = t0_v[...]

    cnt = zero

    def _compact_seg(args):
        cnt, m, segid = args
        pos = cnt + plsc.cumsum(m.astype(jnp.int32)) - 1
        m2 = m & (pos < CAP_SEG)
        plsc.store_scatter(seglist_v, [pos], segid, mask=m2)
        return cnt + plsc.all_reduce_population_count(m2)

    for c in range(SEG_PER_W // L):
        v = segmax_v[pl.ds(c * L, L)]
        m = v >= t0v
        segid = wid * SEG_PER_W + c * L + iota
        cnt = lax.cond(jnp.any(m), _compact_seg, lambda a: a[0], (cnt, m, segid))

    cnt_s = jnp.minimum(jnp.max(cnt, axis=0), CAP_SEG)
    GCH = CAP_SEG // L
    for g in range(GCH):
        @pl.when(g * L < cnt_s)
        def _():
            pltpu.async_copy(
                logits_hbm.at[seglist_v.at[pl.ds(g * L, L)]],
                data_v.at[pl.ds(g * L, L)], sem)
    for g in range(GCH):
        @pl.when(g * L < cnt_s)
        def _():
            pltpu.make_async_copy(
                logits_hbm.at[seglist_v.at[pl.ds(g * L, L)]],
                data_v.at[pl.ds(g * L, L)], sem).wait()

    def _compact_el(args):
        ocnt, m, vals, gidx = args
        pos = ocnt + plsc.cumsum(m.astype(jnp.int32)) - 1
        m2 = m & (pos < CAP_EL)
        plsc.store_scatter(oval_v, [pos], vals, mask=m2)
        plsc.store_scatter(oidx_v, [pos], gidx, mask=m2)
        return ocnt + plsc.all_reduce_population_count(m2)

    def seg_body(i, ocnt):
        i_b = lax.broadcast(i, (L,))
        svec = plsc.load_gather(seglist_v, [i_b])
        for c in range(SEGW // L):
            vals = plsc.load_gather(data_v, [i_b, c * L + iota])
            m = vals >= t0v
            gidx = svec * SEGW + c * L + iota
            ocnt = lax.cond(jnp.any(m), _compact_el, lambda a: a[0],
                            (ocnt, m, vals, gidx))
        return ocnt

    lax.fori_loop(0, cnt_s, seg_body, zero)

    pltpu.sync_copy(oval_v, out_val_hbm.at[wid])
    pltpu.sync_copy(oidx_v, out_idx_hbm.at[wid])


def _sc_compact(logits2d, segmax_flat, t0_vec):
    mesh = plsc.VectorSubcoreMesh(core_axis_name="c", subcore_axis_name="s")
    fn = pl.kernel(
        _sc_compact_body,
        out_type=[
            jax.ShapeDtypeStruct((NW, CAP_EL), jnp.float32),
            jax.ShapeDtypeStruct((NW, CAP_EL), jnp.int32),
        ],
        mesh=mesh,
        compiler_params=pltpu.CompilerParams(needs_layout_passes=False),
        scratch_types=[
            pltpu.VMEM((L,), jnp.float32),
            pltpu.VMEM((SEG_PER_W,), jnp.float32),
            pltpu.VMEM((CAP_SEG,), jnp.int32),
            pltpu.VMEM((CAP_SEG, SEGW), jnp.float32),
            pltpu.VMEM((CAP_EL,), jnp.float32),
            pltpu.VMEM((CAP_EL,), jnp.int32),
            pltpu.SemaphoreType.DMA,
        ],
    )
    return fn(logits2d, segmax_flat, t0_vec)


def kernel(query_set, key_set, assign_mask, pairwise_tti, Wq, bq, Wk, bk, alpha, bias, top_k):
    Q = _project(query_set, Wq, bq)
    Kp = _project(key_set, Wk, bk)
    logits, segmax = _attn(Q, Kp, pairwise_tti, assign_mask, alpha, bias)
    segmax_flat = segmax.reshape(-1)
    t0 = lax.top_k(segmax_flat, K_TOP)[0][K_TOP - 1]
    t0_vec = jnp.full((L,), t0, jnp.float32)
    logits2d = logits.reshape(NSEG, SEGW)
    cand_val, cand_idx = _sc_compact(logits2d, segmax_flat, t0_vec)
    _, bpos = lax.top_k(cand_val.reshape(-1), K_TOP)
    top_flat = cand_idx.reshape(-1)[bpos]
    i_idx = top_flat // N_K
    j_idx = top_flat % N_K
    return jnp.stack([i_idx, j_idx], axis=1)


# speedup vs baseline: 1.5976x; 1.0028x over previous
import math

import jax
import jax.numpy as jnp
from jax import lax
from jax.experimental import pallas as pl
from jax.experimental.pallas import tpu as pltpu
from jax.experimental.pallas import tpu_sc as plsc

D_MODEL = 1024
N_Q = 4096
N_K = 4096
K_TOP = 128
BN = 512
SEGW = 512
NSEG = (N_Q * N_K) // SEGW
NW = 32
SEG_PER_W = NSEG // NW
CAP_SEG = 128
CAP_EL = 512
L = 16


def _proj_body(x_ref, w_ref, b_ref, o_ref):
    o_ref[...] = (
        jax.lax.dot_general(
            x_ref[...], w_ref[...], (((1,), (0,)), ((), ())),
            preferred_element_type=jnp.float32,
        )
        + b_ref[...]
    )


def _project(x, W, b):
    BM_P = 2048
    return pl.pallas_call(
        _proj_body,
        grid=(N_Q // BM_P,),
        in_specs=[
            pl.BlockSpec((BM_P, D_MODEL), lambda i: (i, 0)),
            pl.BlockSpec((D_MODEL, D_MODEL), lambda i: (0, 0)),
            pl.BlockSpec((1, D_MODEL), lambda i: (0, 0)),
        ],
        out_specs=pl.BlockSpec((BM_P, D_MODEL), lambda i: (i, 0)),
        out_shape=jax.ShapeDtypeStruct((N_Q, D_MODEL), jnp.float32),
    )(x, W, b.reshape(1, D_MODEL))


def _attn_body(q_ref, k_ref, tti_ref, m_ref, ab_ref, out_ref, segmax_ref):
    j = pl.program_id(1)
    alpha = ab_ref[0, 0]
    bias = ab_ref[0, 1]
    s = jax.lax.dot_general(
        q_ref[...], k_ref[...], (((1,), (1,)), ((), ())),
        preferred_element_type=jnp.float32,
    )
    s = s * (1.0 / math.sqrt(D_MODEL))
    logits = (bias + s) - alpha * tti_ref[...]
    logits = jnp.where(m_ref[...], logits, jnp.float32(-1.0e9))
    out_ref[...] = logits
    mx = jnp.max(logits, axis=1, keepdims=True)
    col = jax.lax.broadcasted_iota(jnp.int32, (logits.shape[0], N_K // BN), 1)
    prev = jnp.where(col < j, segmax_ref[...], jnp.float32(-jnp.inf))
    segmax_ref[...] = jnp.where(col == j, mx, prev)


def _attn(Q, Kp, tti, mask, alpha, bias):
    ab = jnp.stack([alpha.astype(jnp.float32), bias.astype(jnp.float32)]).reshape(1, 2)
    bm = 2048
    return pl.pallas_call(
        _attn_body,
        grid=(N_Q // bm, N_K // BN),
        in_specs=[
            pl.BlockSpec((bm, D_MODEL), lambda i, j: (i, 0)),
            pl.BlockSpec((BN, D_MODEL), lambda i, j: (j, 0)),
            pl.BlockSpec((bm, BN), lambda i, j: (i, j)),
            pl.BlockSpec((bm, BN), lambda i, j: (i, j)),
            pl.BlockSpec(memory_space=pltpu.SMEM),
        ],
        out_specs=[
            pl.BlockSpec((bm, BN), lambda i, j: (i, j)),
            pl.BlockSpec((bm, N_K // BN), lambda i, j: (i, 0)),
        ],
        out_shape=[
            jax.ShapeDtypeStruct((N_Q, N_K), jnp.float32),
            jax.ShapeDtypeStruct((N_Q, N_K // BN), jnp.float32),
        ],
    )(Q, Kp, tti, mask, ab)


def _sc_compact_body(logits_hbm, segmax_hbm, t0_hbm,
                     out_val_hbm, out_idx_hbm,
                     t0_v, segmax_v, seglist_v, data_v, oval_v, oidx_v, sem):
    wid = lax.axis_index("c") * 16 + lax.axis_index("s")
    iota = lax.iota(jnp.int32, L)
    zero = jnp.zeros((L,), jnp.int32)

    pltpu.sync_copy(t0_hbm, t0_v)
    pltpu.sync_copy(segmax_hbm.at[pl.ds(wid * SEG_PER_W, SEG_PER_W)], segmax_v)

    for c in range(CAP_SEG // L):
        seglist_v[pl.ds(c * L, L)] = zero
    for c in range(CAP_EL // L):
        oval_v[pl.ds(c * L, L)] = jnp.full((L,), -1.0e9, jnp.float32)
        oidx_v[pl.ds(c * L, L)] = zero

    t0v = t0_v[...]

    cnt = zero

    def _compact_seg(args):
        cnt, m, segid = args
        pos = cnt + plsc.cumsum(m.astype(jnp.int32)) - 1
        m2 = m & (pos < CAP_SEG)
        plsc.store_scatter(seglist_v, [pos], segid, mask=m2)
        return cnt + plsc.all_reduce_population_count(m2)

    for c in range(SEG_PER_W // L):
        v = segmax_v[pl.ds(c * L, L)]
        m = v >= t0v
        segid = wid * SEG_PER_W + c * L + iota
        cnt = lax.cond(jnp.any(m), _compact_seg, lambda a: a[0], (cnt, m, segid))

    cnt_s = jnp.minimum(jnp.max(cnt, axis=0), CAP_SEG)
    GCH = CAP_SEG // L
    for g in range(GCH):
        @pl.when(g * L < cnt_s)
        def _():
            pltpu.async_copy(
                logits_hbm.at[seglist_v.at[pl.ds(g * L, L)]],
                data_v.at[pl.ds(g * L, L)], sem)
    for g in range(GCH):
        @pl.when(g * L < cnt_s)
        def _():
            pltpu.make_async_copy(
                logits_hbm.at[seglist_v.at[pl.ds(g * L, L)]],
                data_v.at[pl.ds(g * L, L)], sem).wait()

    def _compact_el(args):
        ocnt, m, vals, gidx = args
        pos = ocnt + plsc.cumsum(m.astype(jnp.int32)) - 1
        m2 = m & (pos < CAP_EL)
        plsc.store_scatter(oval_v, [pos], vals, mask=m2)
        plsc.store_scatter(oidx_v, [pos], gidx, mask=m2)
        return ocnt + plsc.all_reduce_population_count(m2)

    def seg_body(i, ocnt):
        i_b = lax.broadcast(i, (L,))
        svec = plsc.load_gather(seglist_v, [i_b])
        for c in range(SEGW // L):
            vals = plsc.load_gather(data_v, [i_b, c * L + iota])
            m = vals >= t0v
            gidx = svec * SEGW + c * L + iota
            ocnt = lax.cond(jnp.any(m), _compact_el, lambda a: a[0],
                            (ocnt, m, vals, gidx))
        return ocnt

    lax.fori_loop(0, cnt_s, seg_body, zero)

    pltpu.sync_copy(oval_v, out_val_hbm.at[wid])
    pltpu.sync_copy(oidx_v, out_idx_hbm.at[wid])


def _sc_compact(logits2d, segmax_flat, t0_vec):
    mesh = plsc.VectorSubcoreMesh(core_axis_name="c", subcore_axis_name="s")
    fn = pl.kernel(
        _sc_compact_body,
        out_type=[
            jax.ShapeDtypeStruct((NW, CAP_EL), jnp.float32),
            jax.ShapeDtypeStruct((NW, CAP_EL), jnp.int32),
        ],
        mesh=mesh,
        compiler_params=pltpu.CompilerParams(needs_layout_passes=False),
        scratch_types=[
            pltpu.VMEM((L,), jnp.float32),
            pltpu.VMEM((SEG_PER_W,), jnp.float32),
            pltpu.VMEM((CAP_SEG,), jnp.int32),
            pltpu.VMEM((CAP_SEG, SEGW), jnp.float32),
            pltpu.VMEM((CAP_EL,), jnp.float32),
            pltpu.VMEM((CAP_EL,), jnp.int32),
            pltpu.SemaphoreType.DMA,
        ],
    )
    return fn(logits2d, segmax_flat, t0_vec)


def kernel(query_set, key_set, assign_mask, pairwise_tti, Wq, bq, Wk, bk, alpha, bias, top_k):
    Q = _project(query_set, Wq, bq)
    Kp = _project(key_set, Wk, bk)
    logits, segmax = _attn(Q, Kp, pairwise_tti, assign_mask, alpha, bias)
    segmax_flat = segmax.reshape(-1)
    t0 = lax.top_k(segmax_flat, K_TOP)[0][K_TOP - 1]
    t0_vec = jnp.full((L,), t0, jnp.float32)
    logits2d = logits.reshape(NSEG, SEGW)
    cand_val, cand_idx = _sc_compact(logits2d, segmax_flat, t0_vec)
    _, bpos = lax.top_k(cand_val.reshape(-1), K_TOP)
    top_flat = cand_idx.reshape(-1)[bpos]
    i_idx = top_flat // N_K
    j_idx = top_flat % N_K
    return jnp.stack([i_idx, j_idx], axis=1)
